# baseline (speedup 1.0000x reference)
import math

import jax
import jax.numpy as jnp
from jax.experimental import pallas as pl
from jax.experimental.pallas import tpu as pltpu

_B = 128
_V = 100000
_CHUNK = 20000
_NCHUNKS = _V // _CHUNK

_BETA = 0.7
_LOG2E = 1.4426950408889634


def _lse_kernel(l1_ref, l2_ref, val_ref, out_ref, m1, s1, g1, m2, s2, g2):
    pid = pl.program_id(0)

    @pl.when(pid == 0)
    def _init():
        neg_inf = jnp.full((1, _B), -jnp.inf, jnp.float32)
        zero = jnp.zeros((1, _B), jnp.float32)
        m1[...] = neg_inf
        m2[...] = neg_inf
        s1[...] = zero
        s2[...] = zero
        g1[...] = zero
        g2[...] = zero

    def _update(x, m_ref, s_ref):
        m_old = m_ref[...]
        m_new = jnp.maximum(m_old, jnp.max(x, axis=0, keepdims=True))
        m2 = m_new * _LOG2E
        s_ref[...] = s_ref[...] * jnp.exp2(m_old * _LOG2E - m2) + jnp.sum(
            jnp.exp2(x * _LOG2E - m2), axis=0, keepdims=True
        )
        m_ref[...] = m_new

    _update(l1_ref[...], m1, s1)
    _update(l2_ref[...], m2, s2)

    lane = jax.lax.broadcasted_iota(jnp.int32, (1, _B), 1)
    acc1 = [jnp.zeros((1, _B), jnp.float32) for _ in range(4)]
    acc2 = [jnp.zeros((1, _B), jnp.float32) for _ in range(4)]
    for k in range(_B):
        vk = val_ref[0, k]
        row = vk - pid * _CHUNK
        inb = jnp.logical_and(row >= 0, row < _CHUNK)
        rowc = jnp.clip(row, 0, _CHUNK - 1)
        mask = jnp.logical_and(lane == k, inb)
        acc1[k % 4] += jnp.where(mask, l1_ref[pl.ds(rowc, 1), :], 0.0)
        acc2[k % 4] += jnp.where(mask, l2_ref[pl.ds(rowc, 1), :], 0.0)
    g1[...] += acc1[0] + acc1[1] + acc1[2] + acc1[3]
    g2[...] += acc2[0] + acc2[1] + acc2[2] + acc2[3]

    @pl.when(pid == _NCHUNKS - 1)
    def _finish():
        lp1 = g1[...] - m1[...] - jnp.log(s1[...]) + math.log(_BETA)
        lp2 = g2[...] - m2[...] - jnp.log(s2[...]) + math.log(1.0 - _BETA)
        mx = jnp.maximum(lp1, lp2)
        out_ref[...] = mx + jnp.log(jnp.exp(lp1 - mx) + jnp.exp(lp2 - mx))


@jax.jit
def kernel(logits_1, logits_2, value):
    lt1 = logits_1.T
    lt2 = logits_2.T
    val2d = value.astype(jnp.int32).reshape(1, _B)
    return pl.pallas_call(
        _lse_kernel,
        grid=(_NCHUNKS,),
        in_specs=[
            pl.BlockSpec((_CHUNK, _B), lambda i: (i, 0)),
            pl.BlockSpec((_CHUNK, _B), lambda i: (i, 0)),
            pl.BlockSpec(memory_space=pltpu.SMEM),
        ],
        out_specs=pl.BlockSpec((1, _B), lambda i: (0, 0)),
        out_shape=jax.ShapeDtypeStruct((1, _B), jnp.float32),
        scratch_shapes=[pltpu.VMEM((1, _B), jnp.float32) for _ in range(6)],
    )(lt1, lt2, val2d)


# speedup vs baseline: 1.0139x; 1.0139x over previous
import math

import jax
import jax.numpy as jnp
from jax.experimental import pallas as pl
from jax.experimental.pallas import tpu as pltpu

_B = 128
_V = 100000
_CHUNK = 25000
_NCHUNKS = _V // _CHUNK

_BETA = 0.7
_LOG2E = 1.4426950408889634


def _lse_kernel(l1_ref, l2_ref, val_ref, out_ref, m1, s1, g1, m2, s2, g2):
    pid = pl.program_id(0)

    @pl.when(pid == 0)
    def _init():
        neg_inf = jnp.full((1, _B), -jnp.inf, jnp.float32)
        zero = jnp.zeros((1, _B), jnp.float32)
        m1[...] = neg_inf
        m2[...] = neg_inf
        s1[...] = zero
        s2[...] = zero
        g1[...] = zero
        g2[...] = zero

    def _update(x, m_ref, s_ref):
        m_old = m_ref[...]
        m_new = jnp.maximum(m_old, jnp.max(x, axis=0, keepdims=True))
        s_ref[...] = s_ref[...] * jnp.exp(m_old - m_new) + jnp.sum(
            jnp.exp(x - m_new), axis=0, keepdims=True
        )
        m_ref[...] = m_new

    _update(l1_ref[...], m1, s1)
    _update(l2_ref[...], m2, s2)

    lane = jax.lax.broadcasted_iota(jnp.int32, (1, _B), 1)
    acc1 = [jnp.zeros((1, _B), jnp.float32) for _ in range(4)]
    acc2 = [jnp.zeros((1, _B), jnp.float32) for _ in range(4)]
    for k in range(_B):
        vk = val_ref[0, k]
        row = vk - pid * _CHUNK
        inb = jnp.logical_and(row >= 0, row < _CHUNK)
        rowc = jnp.clip(row, 0, _CHUNK - 1)
        mask = jnp.logical_and(lane == k, inb)
        acc1[k % 4] += jnp.where(mask, l1_ref[pl.ds(rowc, 1), :], 0.0)
        acc2[k % 4] += jnp.where(mask, l2_ref[pl.ds(rowc, 1), :], 0.0)
    g1[...] += acc1[0] + acc1[1] + acc1[2] + acc1[3]
    g2[...] += acc2[0] + acc2[1] + acc2[2] + acc2[3]

    @pl.when(pid == _NCHUNKS - 1)
    def _finish():
        lp1 = g1[...] - m1[...] - jnp.log(s1[...]) + math.log(_BETA)
        lp2 = g2[...] - m2[...] - jnp.log(s2[...]) + math.log(1.0 - _BETA)
        mx = jnp.maximum(lp1, lp2)
        out_ref[...] = mx + jnp.log(jnp.exp(lp1 - mx) + jnp.exp(lp2 - mx))


@jax.jit
def kernel(logits_1, logits_2, value):
    lt1 = logits_1.T
    lt2 = logits_2.T
    val2d = value.astype(jnp.int32).reshape(1, _B)
    return pl.pallas_call(
        _lse_kernel,
        grid=(_NCHUNKS,),
        in_specs=[
            pl.BlockSpec((_CHUNK, _B), lambda i: (i, 0)),
            pl.BlockSpec((_CHUNK, _B), lambda i: (i, 0)),
            pl.BlockSpec(memory_space=pltpu.SMEM),
        ],
        out_specs=pl.BlockSpec((1, _B), lambda i: (0, 0)),
        out_shape=jax.ShapeDtypeStruct((1, _B), jnp.float32),
        scratch_shapes=[pltpu.VMEM((1, _B), jnp.float32) for _ in range(6)],
    )(lt1, lt2, val2d)


# speedup vs baseline: 1.0305x; 1.0164x over previous
import math

import jax
import jax.numpy as jnp
from jax.experimental import pallas as pl
from jax.experimental.pallas import tpu as pltpu

_B = 128
_V = 100000
_CHUNK = 20000
_NCHUNKS = _V // _CHUNK

_BETA = 0.7
_LOG2E = 1.4426950408889634


def _lse_kernel(l1_ref, l2_ref, val_ref, out_ref, m1, s1, g1, m2, s2, g2):
    pid = pl.program_id(0)

    @pl.when(pid == 0)
    def _init():
        neg_inf = jnp.full((1, _B), -jnp.inf, jnp.float32)
        zero = jnp.zeros((1, _B), jnp.float32)
        m1[...] = neg_inf
        m2[...] = neg_inf
        s1[...] = zero
        s2[...] = zero
        g1[...] = zero
        g2[...] = zero

    def _update(x, m_ref, s_ref):
        m_old = m_ref[...]
        m_new = jnp.maximum(m_old, jnp.max(x, axis=0, keepdims=True))
        s_ref[...] = s_ref[...] * jnp.exp(m_old - m_new) + jnp.sum(
            jnp.exp(x - m_new), axis=0, keepdims=True
        )
        m_ref[...] = m_new

    _update(l1_ref[...], m1, s1)
    _update(l2_ref[...], m2, s2)

    lane = jax.lax.broadcasted_iota(jnp.int32, (1, _B), 1)
    acc1 = [jnp.zeros((1, _B), jnp.float32) for _ in range(4)]
    acc2 = [jnp.zeros((1, _B), jnp.float32) for _ in range(4)]
    for k in range(_B):
        vk = val_ref[0, k]
        row = vk - pid * _CHUNK
        inb = jnp.logical_and(row >= 0, row < _CHUNK)
        rowc = jnp.clip(row, 0, _CHUNK - 1)
        mask = jnp.logical_and(lane == k, inb)
        acc1[k % 4] += jnp.where(mask, l1_ref[pl.ds(rowc, 1), :], 0.0)
        acc2[k % 4] += jnp.where(mask, l2_ref[pl.ds(rowc, 1), :], 0.0)
    g1[...] += acc1[0] + acc1[1] + acc1[2] + acc1[3]
    g2[...] += acc2[0] + acc2[1] + acc2[2] + acc2[3]

    @pl.when(pid == _NCHUNKS - 1)
    def _finish():
        lp1 = g1[...] - m1[...] - jnp.log(s1[...]) + math.log(_BETA)
        lp2 = g2[...] - m2[...] - jnp.log(s2[...]) + math.log(1.0 - _BETA)
        mx = jnp.maximum(lp1, lp2)
        out_ref[...] = mx + jnp.log(jnp.exp(lp1 - mx) + jnp.exp(lp2 - mx))


@jax.jit
def kernel(logits_1, logits_2, value):
    lt1 = logits_1.T
    lt2 = logits_2.T
    val2d = value.astype(jnp.int32).reshape(1, _B)
    return pl.pallas_call(
        _lse_kernel,
        grid=(_NCHUNKS,),
        in_specs=[
            pl.BlockSpec((_CHUNK, _B), lambda i: (i, 0)),
            pl.BlockSpec((_CHUNK, _B), lambda i: (i, 0)),
            pl.BlockSpec(memory_space=pltpu.SMEM),
        ],
        out_specs=pl.BlockSpec((1, _B), lambda i: (0, 0)),
        out_shape=jax.ShapeDtypeStruct((1, _B), jnp.float32),
        scratch_shapes=[pltpu.VMEM((1, _B), jnp.float32) for _ in range(6)],
    )(lt1, lt2, val2d)


# speedup vs baseline: 1.1813x; 1.1463x over previous
import math

import jax
import jax.numpy as jnp
from jax.experimental import pallas as pl
from jax.experimental.pallas import tpu as pltpu

_B = 128
_V = 100000
_CHUNK = 20000
_HALF = _CHUNK // 2
_NCHUNKS = _V // _CHUNK

_BETA = 0.7


def _merge(m_a, s_a, m_b, s_b):
    m = jnp.maximum(m_a, m_b)
    return m, s_a * jnp.exp(m_a - m) + s_b * jnp.exp(m_b - m)


def _lse_kernel(
    l1_ref, l2_ref, val_ref, out_ref,
    m1a, s1a, m1b, s1b, m2a, s2a, m2b, s2b, g1, g2,
):
    pid = pl.program_id(0)

    @pl.when(pid == 0)
    def _init():
        neg_inf = jnp.full((1, _B), -jnp.inf, jnp.float32)
        zero = jnp.zeros((1, _B), jnp.float32)
        for m_ref in (m1a, m1b, m2a, m2b):
            m_ref[...] = neg_inf
        for s_ref in (s1a, s1b, s2a, s2b, g1, g2):
            s_ref[...] = zero

    def _update(x, m_ref, s_ref):
        m_old = m_ref[...]
        m_new = jnp.maximum(m_old, jnp.max(x, axis=0, keepdims=True))
        s_ref[...] = s_ref[...] * jnp.exp(m_old - m_new) + jnp.sum(
            jnp.exp(x - m_new), axis=0, keepdims=True
        )
        m_ref[...] = m_new

    _update(l1_ref[pl.ds(0, _HALF), :], m1a, s1a)
    _update(l1_ref[pl.ds(_HALF, _HALF), :], m1b, s1b)
    _update(l2_ref[pl.ds(0, _HALF), :], m2a, s2a)
    _update(l2_ref[pl.ds(_HALF, _HALF), :], m2b, s2b)

    lane = jax.lax.broadcasted_iota(jnp.int32, (1, _B), 1)
    acc1 = [jnp.zeros((1, _B), jnp.float32) for _ in range(4)]
    acc2 = [jnp.zeros((1, _B), jnp.float32) for _ in range(4)]
    for k in range(_B):
        vk = val_ref[0, k]
        row = vk - pid * _CHUNK
        inb = jnp.logical_and(row >= 0, row < _CHUNK)
        rowc = jnp.clip(row, 0, _CHUNK - 1)
        mask = jnp.logical_and(lane == k, inb)
        acc1[k % 4] += jnp.where(mask, l1_ref[pl.ds(rowc, 1), :], 0.0)
        acc2[k % 4] += jnp.where(mask, l2_ref[pl.ds(rowc, 1), :], 0.0)
    g1[...] += acc1[0] + acc1[1] + acc1[2] + acc1[3]
    g2[...] += acc2[0] + acc2[1] + acc2[2] + acc2[3]

    @pl.when(pid == _NCHUNKS - 1)
    def _finish():
        m1, s1 = _merge(m1a[...], s1a[...], m1b[...], s1b[...])
        m2, s2 = _merge(m2a[...], s2a[...], m2b[...], s2b[...])
        lp1 = g1[...] - m1 - jnp.log(s1) + math.log(_BETA)
        lp2 = g2[...] - m2 - jnp.log(s2) + math.log(1.0 - _BETA)
        mx = jnp.maximum(lp1, lp2)
        out_ref[...] = mx + jnp.log(jnp.exp(lp1 - mx) + jnp.exp(lp2 - mx))


@jax.jit
def kernel(logits_1, logits_2, value):
    lt1 = logits_1.T
    lt2 = logits_2.T
    val2d = value.astype(jnp.int32).reshape(1, _B)
    return pl.pallas_call(
        _lse_kernel,
        grid=(_NCHUNKS,),
        in_specs=[
            pl.BlockSpec((_CHUNK, _B), lambda i: (i, 0)),
            pl.BlockSpec((_CHUNK, _B), lambda i: (i, 0)),
            pl.BlockSpec(memory_space=pltpu.SMEM),
        ],
        out_specs=pl.BlockSpec((1, _B), lambda i: (0, 0)),
        out_shape=jax.ShapeDtypeStruct((1, _B), jnp.float32),
        scratch_shapes=[pltpu.VMEM((1, _B), jnp.float32) for _ in range(10)],
    )(lt1, lt2, val2d)


# speedup vs baseline: 1.2403x; 1.0499x over previous
import math

import jax
import jax.numpy as jnp
from jax.experimental import pallas as pl
from jax.experimental.pallas import tpu as pltpu

_B = 128
_V = 100000
_CHUNK = 20000
_QUARTER = _CHUNK // 4
_NCHUNKS = _V // _CHUNK

_BETA = 0.7


def _merge(m_a, s_a, m_b, s_b):
    m = jnp.maximum(m_a, m_b)
    return m, s_a * jnp.exp(m_a - m) + s_b * jnp.exp(m_b - m)


def _lse_kernel(
    l1_ref, l2_ref, val_ref, out_ref,
    m1a, s1a, m1b, s1b, m1c, s1c, m1d, s1d,
    m2a, s2a, m2b, s2b, m2c, s2c, m2d, s2d, g1, g2,
):
    pid = pl.program_id(0)

    @pl.when(pid == 0)
    def _init():
        neg_inf = jnp.full((1, _B), -jnp.inf, jnp.float32)
        zero = jnp.zeros((1, _B), jnp.float32)
        for m_ref in (m1a, m1b, m1c, m1d, m2a, m2b, m2c, m2d):
            m_ref[...] = neg_inf
        for s_ref in (s1a, s1b, s1c, s1d, s2a, s2b, s2c, s2d, g1, g2):
            s_ref[...] = zero

    def _update(x, m_ref, s_ref):
        m_old = m_ref[...]
        m_new = jnp.maximum(m_old, jnp.max(x, axis=0, keepdims=True))
        s_ref[...] = s_ref[...] * jnp.exp(m_old - m_new) + jnp.sum(
            jnp.exp(x - m_new), axis=0, keepdims=True
        )
        m_ref[...] = m_new

    for j, (m_ref, s_ref) in enumerate(
        ((m1a, s1a), (m1b, s1b), (m1c, s1c), (m1d, s1d))
    ):
        _update(l1_ref[pl.ds(j * _QUARTER, _QUARTER), :], m_ref, s_ref)
    for j, (m_ref, s_ref) in enumerate(
        ((m2a, s2a), (m2b, s2b), (m2c, s2c), (m2d, s2d))
    ):
        _update(l2_ref[pl.ds(j * _QUARTER, _QUARTER), :], m_ref, s_ref)

    lane = jax.lax.broadcasted_iota(jnp.int32, (1, _B), 1)
    acc1 = [jnp.zeros((1, _B), jnp.float32) for _ in range(4)]
    acc2 = [jnp.zeros((1, _B), jnp.float32) for _ in range(4)]
    for k in range(_B):
        vk = val_ref[0, k]
        row = vk - pid * _CHUNK
        inb = jnp.logical_and(row >= 0, row < _CHUNK)
        rowc = jnp.clip(row, 0, _CHUNK - 1)
        mask = jnp.logical_and(lane == k, inb)
        acc1[k % 4] += jnp.where(mask, l1_ref[pl.ds(rowc, 1), :], 0.0)
        acc2[k % 4] += jnp.where(mask, l2_ref[pl.ds(rowc, 1), :], 0.0)
    g1[...] += acc1[0] + acc1[1] + acc1[2] + acc1[3]
    g2[...] += acc2[0] + acc2[1] + acc2[2] + acc2[3]

    @pl.when(pid == _NCHUNKS - 1)
    def _finish():
        m1, s1 = _merge(*_merge(m1a[...], s1a[...], m1b[...], s1b[...]),
                        *_merge(m1c[...], s1c[...], m1d[...], s1d[...]))
        m2, s2 = _merge(*_merge(m2a[...], s2a[...], m2b[...], s2b[...]),
                        *_merge(m2c[...], s2c[...], m2d[...], s2d[...]))
        lp1 = g1[...] - m1 - jnp.log(s1) + math.log(_BETA)
        lp2 = g2[...] - m2 - jnp.log(s2) + math.log(1.0 - _BETA)
        mx = jnp.maximum(lp1, lp2)
        out_ref[...] = mx + jnp.log(jnp.exp(lp1 - mx) + jnp.exp(lp2 - mx))


@jax.jit
def kernel(logits_1, logits_2, value):
    lt1 = logits_1.T
    lt2 = logits_2.T
    val2d = value.astype(jnp.int32).reshape(1, _B)
    return pl.pallas_call(
        _lse_kernel,
        grid=(_NCHUNKS,),
        in_specs=[
            pl.BlockSpec((_CHUNK, _B), lambda i: (i, 0)),
            pl.BlockSpec((_CHUNK, _B), lambda i: (i, 0)),
            pl.BlockSpec(memory_space=pltpu.SMEM),
        ],
        out_specs=pl.BlockSpec((1, _B), lambda i: (0, 0)),
        out_shape=jax.ShapeDtypeStruct((1, _B), jnp.float32),
        scratch_shapes=[pltpu.VMEM((1, _B), jnp.float32) for _ in range(18)],
    )(lt1, lt2, val2d)


# speedup vs baseline: 1.2849x; 1.0360x over previous
import math

import jax
import jax.numpy as jnp
from jax.experimental import pallas as pl
from jax.experimental.pallas import tpu as pltpu

_B = 128
_V = 100000
_CHUNK = 20000
_NSPLIT = 8
_SUB = _CHUNK // _NSPLIT
_NCHUNKS = _V // _CHUNK

_BETA = 0.7


def _merge(m_a, s_a, m_b, s_b):
    m = jnp.maximum(m_a, m_b)
    return m, s_a * jnp.exp(m_a - m) + s_b * jnp.exp(m_b - m)


def _lse_kernel(l1_ref, l2_ref, val_ref, out_ref, *scratch):
    ms1 = [(scratch[2 * j], scratch[2 * j + 1]) for j in range(_NSPLIT)]
    ms2 = [(scratch[2 * _NSPLIT + 2 * j], scratch[2 * _NSPLIT + 2 * j + 1])
           for j in range(_NSPLIT)]
    g1, g2 = scratch[4 * _NSPLIT], scratch[4 * _NSPLIT + 1]
    pid = pl.program_id(0)

    @pl.when(pid == 0)
    def _init():
        neg_inf = jnp.full((1, _B), -jnp.inf, jnp.float32)
        zero = jnp.zeros((1, _B), jnp.float32)
        for m_ref, s_ref in ms1 + ms2:
            m_ref[...] = neg_inf
            s_ref[...] = zero
        g1[...] = zero
        g2[...] = zero

    def _update(x, m_ref, s_ref):
        m_old = m_ref[...]
        m_new = jnp.maximum(m_old, jnp.max(x, axis=0, keepdims=True))
        s_ref[...] = s_ref[...] * jnp.exp(m_old - m_new) + jnp.sum(
            jnp.exp(x - m_new), axis=0, keepdims=True
        )
        m_ref[...] = m_new

    for j, (m_ref, s_ref) in enumerate(ms1):
        _update(l1_ref[pl.ds(j * _SUB, _SUB), :], m_ref, s_ref)
    for j, (m_ref, s_ref) in enumerate(ms2):
        _update(l2_ref[pl.ds(j * _SUB, _SUB), :], m_ref, s_ref)

    lane = jax.lax.broadcasted_iota(jnp.int32, (1, _B), 1)
    acc1 = [jnp.zeros((1, _B), jnp.float32) for _ in range(4)]
    acc2 = [jnp.zeros((1, _B), jnp.float32) for _ in range(4)]
    for k in range(_B):
        vk = val_ref[0, k]
        row = vk - pid * _CHUNK
        inb = jnp.logical_and(row >= 0, row < _CHUNK)
        rowc = jnp.clip(row, 0, _CHUNK - 1)
        mask = jnp.logical_and(lane == k, inb)
        acc1[k % 4] += jnp.where(mask, l1_ref[pl.ds(rowc, 1), :], 0.0)
        acc2[k % 4] += jnp.where(mask, l2_ref[pl.ds(rowc, 1), :], 0.0)
    g1[...] += acc1[0] + acc1[1] + acc1[2] + acc1[3]
    g2[...] += acc2[0] + acc2[1] + acc2[2] + acc2[3]

    @pl.when(pid == _NCHUNKS - 1)
    def _finish():
        def _merge_all(ms):
            parts = [(m_ref[...], s_ref[...]) for m_ref, s_ref in ms]
            while len(parts) > 1:
                parts = [_merge(*parts[i], *parts[i + 1])
                         for i in range(0, len(parts), 2)]
            return parts[0]

        m1, s1 = _merge_all(ms1)
        m2, s2 = _merge_all(ms2)
        lp1 = g1[...] - m1 - jnp.log(s1) + math.log(_BETA)
        lp2 = g2[...] - m2 - jnp.log(s2) + math.log(1.0 - _BETA)
        mx = jnp.maximum(lp1, lp2)
        out_ref[...] = mx + jnp.log(jnp.exp(lp1 - mx) + jnp.exp(lp2 - mx))


@jax.jit
def kernel(logits_1, logits_2, value):
    lt1 = logits_1.T
    lt2 = logits_2.T
    val2d = value.astype(jnp.int32).reshape(1, _B)
    return pl.pallas_call(
        _lse_kernel,
        grid=(_NCHUNKS,),
        in_specs=[
            pl.BlockSpec((_CHUNK, _B), lambda i: (i, 0)),
            pl.BlockSpec((_CHUNK, _B), lambda i: (i, 0)),
            pl.BlockSpec(memory_space=pltpu.SMEM),
        ],
        out_specs=pl.BlockSpec((1, _B), lambda i: (0, 0)),
        out_shape=jax.ShapeDtypeStruct((1, _B), jnp.float32),
        scratch_shapes=[pltpu.VMEM((1, _B), jnp.float32) for _ in range(4 * _NSPLIT + 2)],
    )(lt1, lt2, val2d)


# speedup vs baseline: 1.2862x; 1.0011x over previous
import math

import jax
import jax.numpy as jnp
from jax.experimental import pallas as pl
from jax.experimental.pallas import tpu as pltpu

_B = 128
_V = 100000
_CHUNK = 20000
_NSPLIT = 16
_SUB = _CHUNK // _NSPLIT
_NCHUNKS = _V // _CHUNK

_BETA = 0.7


def _merge(m_a, s_a, m_b, s_b):
    m = jnp.maximum(m_a, m_b)
    return m, s_a * jnp.exp(m_a - m) + s_b * jnp.exp(m_b - m)


def _lse_kernel(l1_ref, l2_ref, val_ref, out_ref, *scratch):
    ms1 = [(scratch[2 * j], scratch[2 * j + 1]) for j in range(_NSPLIT)]
    ms2 = [(scratch[2 * _NSPLIT + 2 * j], scratch[2 * _NSPLIT + 2 * j + 1])
           for j in range(_NSPLIT)]
    g1, g2 = scratch[4 * _NSPLIT], scratch[4 * _NSPLIT + 1]
    pid = pl.program_id(0)

    @pl.when(pid == 0)
    def _init():
        neg_inf = jnp.full((1, _B), -jnp.inf, jnp.float32)
        zero = jnp.zeros((1, _B), jnp.float32)
        for m_ref, s_ref in ms1 + ms2:
            m_ref[...] = neg_inf
            s_ref[...] = zero
        g1[...] = zero
        g2[...] = zero

    def _update(x, m_ref, s_ref):
        m_old = m_ref[...]
        m_new = jnp.maximum(m_old, jnp.max(x, axis=0, keepdims=True))
        s_ref[...] = s_ref[...] * jnp.exp(m_old - m_new) + jnp.sum(
            jnp.exp(x - m_new), axis=0, keepdims=True
        )
        m_ref[...] = m_new

    for j, (m_ref, s_ref) in enumerate(ms1):
        _update(l1_ref[pl.ds(j * _SUB, _SUB), :], m_ref, s_ref)
    for j, (m_ref, s_ref) in enumerate(ms2):
        _update(l2_ref[pl.ds(j * _SUB, _SUB), :], m_ref, s_ref)

    lane = jax.lax.broadcasted_iota(jnp.int32, (1, _B), 1)
    acc1 = [jnp.zeros((1, _B), jnp.float32) for _ in range(4)]
    acc2 = [jnp.zeros((1, _B), jnp.float32) for _ in range(4)]
    for k in range(_B):
        vk = val_ref[0, k]
        row = vk - pid * _CHUNK
        inb = jnp.logical_and(row >= 0, row < _CHUNK)
        rowc = jnp.clip(row, 0, _CHUNK - 1)
        mask = jnp.logical_and(lane == k, inb)
        acc1[k % 4] += jnp.where(mask, l1_ref[pl.ds(rowc, 1), :], 0.0)
        acc2[k % 4] += jnp.where(mask, l2_ref[pl.ds(rowc, 1), :], 0.0)
    g1[...] += acc1[0] + acc1[1] + acc1[2] + acc1[3]
    g2[...] += acc2[0] + acc2[1] + acc2[2] + acc2[3]

    @pl.when(pid == _NCHUNKS - 1)
    def _finish():
        def _merge_all(ms):
            parts = [(m_ref[...], s_ref[...]) for m_ref, s_ref in ms]
            while len(parts) > 1:
                parts = [_merge(*parts[i], *parts[i + 1])
                         for i in range(0, len(parts), 2)]
            return parts[0]

        m1, s1 = _merge_all(ms1)
        m2, s2 = _merge_all(ms2)
        lp1 = g1[...] - m1 - jnp.log(s1) + math.log(_BETA)
        lp2 = g2[...] - m2 - jnp.log(s2) + math.log(1.0 - _BETA)
        mx = jnp.maximum(lp1, lp2)
        out_ref[...] = mx + jnp.log(jnp.exp(lp1 - mx) + jnp.exp(lp2 - mx))


@jax.jit
def kernel(logits_1, logits_2, value):
    lt1 = logits_1.T
    lt2 = logits_2.T
    val2d = value.astype(jnp.int32).reshape(1, _B)
    return pl.pallas_call(
        _lse_kernel,
        grid=(_NCHUNKS,),
        in_specs=[
            pl.BlockSpec((_CHUNK, _B), lambda i: (i, 0)),
            pl.BlockSpec((_CHUNK, _B), lambda i: (i, 0)),
            pl.BlockSpec(memory_space=pltpu.SMEM),
        ],
        out_specs=pl.BlockSpec((1, _B), lambda i: (0, 0)),
        out_shape=jax.ShapeDtypeStruct((1, _B), jnp.float32),
        scratch_shapes=[pltpu.VMEM((1, _B), jnp.float32) for _ in range(4 * _NSPLIT + 2)],
    )(lt1, lt2, val2d)
